# baseline (speedup 1.0000x reference)
import functools

import jax
import jax.numpy as jnp
from jax import lax
from jax.experimental import pallas as pl
from jax.experimental.pallas import tpu as pltpu
from jax.experimental.pallas import tpu_sc as plsc

NC = 2
NS = 16
NW = NC * NS
LANES = 16


@functools.partial(jax.jit, static_argnums=(3, 4, 5))
def _sc_embed(inp2, token_table, pos_block, b, l, h):
    rb = b // NW
    half = l // 2
    hc = h // LANES

    mesh = plsc.VectorSubcoreMesh(core_axis_name="c", subcore_axis_name="s")

    h2 = 2 * h

    def body(inp_hbm, tok_hbm, pos_hbm, out_hbm,
             idx_v, rows0, rows1, outv0, outv1, pos_v,
             sem_g, sem_o):
        wid = lax.axis_index("s") * NC + lax.axis_index("c")
        base = wid * rb
        pltpu.sync_copy(pos_hbm, pos_v)
        pltpu.sync_copy(inp_hbm.at[pl.ds(base, rb)], idx_v)

        rows_v = (rows0, rows1)
        out_v = (outv0, outv1)

        def issue_gather(i, s):
            pltpu.async_copy(
                tok_hbm.at[idx_v.at[i, 0]], rows_v[s].at[pl.ds(0, half)],
                sem_g)
            pltpu.async_copy(
                tok_hbm.at[idx_v.at[i, 1]], rows_v[s].at[pl.ds(half, half)],
                sem_g)

        def wait_gather(i, s):
            pltpu.make_async_copy(
                tok_hbm.at[idx_v.at[i, 0]], rows_v[s].at[pl.ds(0, half)],
                sem_g).wait()
            pltpu.make_async_copy(
                tok_hbm.at[idx_v.at[i, 1]], rows_v[s].at[pl.ds(half, half)],
                sem_g).wait()

        def compute(s):
            def tok_body(t, _):
                o = t * h
                for c in range(hc):
                    sl = pl.ds(c * LANES, LANES)
                    out_v[s][pl.ds(o + c * LANES, LANES)] = (
                        rows_v[s][t, sl] + pos_v[t, sl])
                return ()
            lax.fori_loop(0, l, tok_body, (), unroll=8)

        def issue_out(i, s):
            pltpu.async_copy(out_v[s], out_hbm.at[base + i], sem_o)

        def wait_out(i, s):
            pltpu.make_async_copy(out_v[s], out_hbm.at[base + i], sem_o).wait()

        issue_gather(0, 0)

        def step(i, s, issue_ahead, drain_out):
            wait_gather(i, s)
            if issue_ahead:
                issue_gather(i + 1, 1 - s)
            if drain_out:
                wait_out(i - 2, s)
            compute(s)
            issue_out(i, s)

        step(0, 0, True, False)
        step(1, 1, True, False)

        def pair_body(j, _):
            i0 = 2 * j + 2
            step(i0, 0, True, True)
            step(i0 + 1, 1, True, True)
            return ()

        lax.fori_loop(0, (rb - 2) // 2 - 1, pair_body, ())

        step(rb - 2, 0, True, True)
        step(rb - 1, 1, False, True)
        wait_out(rb - 2, 0)
        wait_out(rb - 1, 1)

    call = pl.kernel(
        body,
        out_type=jax.ShapeDtypeStruct((b, l * h), jnp.float32),
        mesh=mesh,
        scratch_types=[
            pltpu.VMEM((rb, 2, half), jnp.int32),
            pltpu.VMEM((l, h2), jnp.float32),
            pltpu.VMEM((l, h2), jnp.float32),
            pltpu.VMEM((l * h,), jnp.float32),
            pltpu.VMEM((l * h,), jnp.float32),
            pltpu.VMEM((l, h), jnp.float32),
            pltpu.SemaphoreType.DMA,
            pltpu.SemaphoreType.DMA,
        ],
        compiler_params=pltpu.CompilerParams(use_tc_tiling_on_sc=False),
    )
    return call(inp2, token_table, pos_block)


def kernel(input, token_table, pos_table):
    b, l = input.shape
    h = token_table.shape[1]
    inp2 = input.reshape(b, 2, l // 2)
    pos_block = lax.slice(pos_table, (1, 0), (1 + l, h))
    tokdup = jnp.concatenate([token_table, token_table], axis=1)
    out = _sc_embed(inp2, tokdup, pos_block, b, l, h)
    return out.reshape(b, l, h)


# speedup vs baseline: 1.0622x; 1.0622x over previous
import functools

import jax
import jax.numpy as jnp
from jax import lax
from jax.experimental import pallas as pl
from jax.experimental.pallas import tpu as pltpu
from jax.experimental.pallas import tpu_sc as plsc

NC = 2
NS = 16
NW = NC * NS
LANES = 16


@functools.partial(jax.jit, static_argnums=(3, 4, 5))
def _sc_embed(inp2, token_table, pos_block, b, l, h):
    rb = b // NW
    half = l // 2
    hc = h // LANES

    mesh = plsc.VectorSubcoreMesh(core_axis_name="c", subcore_axis_name="s")

    h2 = 2 * h

    def body(inp_hbm, tok_hbm, pos_hbm, out_hbm,
             idx_v, rows0, rows1, outv0, outv1, pos_v,
             sem_g, sem_o):
        wid = lax.axis_index("s") * NC + lax.axis_index("c")
        base = wid * rb
        pltpu.sync_copy(pos_hbm, pos_v)
        pltpu.sync_copy(inp_hbm.at[pl.ds(base, rb)], idx_v)

        rows_v = (rows0, rows1)
        out_v = (outv0, outv1)

        def issue_gather(i, s):
            pltpu.async_copy(tok_hbm.at[idx_v.at[i]], rows_v[s], sem_g)

        def wait_gather(i, s):
            pltpu.make_async_copy(
                tok_hbm.at[idx_v.at[i]], rows_v[s], sem_g).wait()

        def compute(s):
            def tok_body(t, _):
                o = t * h
                for c in range(hc):
                    sl = pl.ds(c * LANES, LANES)
                    out_v[s][pl.ds(o + c * LANES, LANES)] = (
                        rows_v[s][0, t, sl] + pos_v[t, sl])
                return ()
            lax.fori_loop(0, l, tok_body, (), unroll=8)

        def issue_out(i, s):
            pltpu.async_copy(out_v[s], out_hbm.at[base + i], sem_o)

        def wait_out(i, s):
            pltpu.make_async_copy(out_v[s], out_hbm.at[base + i], sem_o).wait()

        issue_gather(0, 0)

        def step(i, s, issue_ahead, drain_out):
            wait_gather(i, s)
            if issue_ahead:
                issue_gather(i + 1, 1 - s)
            if drain_out:
                wait_out(i - 2, s)
            compute(s)
            issue_out(i, s)

        step(0, 0, True, False)
        step(1, 1, True, False)

        def pair_body(j, _):
            i0 = 2 * j + 2
            step(i0, 0, True, True)
            step(i0 + 1, 1, True, True)
            return ()

        lax.fori_loop(0, (rb - 2) // 2 - 1, pair_body, ())

        step(rb - 2, 0, True, True)
        step(rb - 1, 1, False, True)
        wait_out(rb - 2, 0)
        wait_out(rb - 1, 1)

    call = pl.kernel(
        body,
        out_type=jax.ShapeDtypeStruct((b, l * h), jnp.float32),
        mesh=mesh,
        scratch_types=[
            pltpu.VMEM((rb, 1, l), jnp.int32),
            pltpu.VMEM((1, l, h), jnp.float32),
            pltpu.VMEM((1, l, h), jnp.float32),
            pltpu.VMEM((l * h,), jnp.float32),
            pltpu.VMEM((l * h,), jnp.float32),
            pltpu.VMEM((l, h), jnp.float32),
            pltpu.SemaphoreType.DMA,
            pltpu.SemaphoreType.DMA,
        ],
        compiler_params=pltpu.CompilerParams(use_tc_tiling_on_sc=False),
    )
    return call(inp2, token_table, pos_block)


def kernel(input, token_table, pos_table):
    b, l = input.shape
    h = token_table.shape[1]
    inp2 = input.reshape(b, 1, l)
    pos_block = lax.slice(pos_table, (1, 0), (1 + l, h))
    tok3 = token_table.reshape(1, token_table.shape[0], h)
    out = _sc_embed(inp2, tok3, pos_block, b, l, h)
    return out.reshape(b, l, h)
